# baseline (speedup 1.0000x reference)
import jax
import jax.numpy as jnp
from jax import lax
from jax.experimental import pallas as pl
from jax.experimental.pallas import tpu as pltpu
from jax.experimental.pallas import tpu_sc as plsc

N = 10000
D = 128
E = 320000

NC = 2
NS = 16
LANE = 128
N_PAD = 10240
ROWS_PER_TILE = N_PAD // NS
WB = ROWS_PER_TILE // 128
K = 80
E_PAD = NC * NS * K * LANE
F32 = jnp.float32


def _sc_scatter():
    mesh = plsc.VectorSubcoreMesh(core_axis_name="c", subcore_axis_name="s")

    def body(table, src3, z128, acc_out, acc, sd0, sd1, rows,
             sem, sem2, isem0, isem1):
        c = lax.axis_index("c")
        s = lax.axis_index("s")
        wid = c * NS + s
        base = s * ROWS_PER_TILE

        pltpu.sync_copy(z128, rows)
        for j in range(WB):
            pltpu.sync_copy(rows, acc.at[pl.ds(base + j * 128, 128)])
        plsc.subcore_barrier()

        pltpu.sync_copy(src3.at[wid, 0], sd0)

        def pipe(g, carry):
            cc = g * 2
            p1 = pltpu.async_copy(src3.at[wid, cc + 1], sd1, isem1)
            pltpu.async_copy(table.at[sd0.at[0]], rows, sem).wait()
            pltpu.async_copy(rows, acc.at[sd0.at[1]], sem2,
                             add=True).wait()
            p1.wait()
            p0 = pltpu.async_copy(src3.at[wid, cc + 2], sd0, isem0)
            pltpu.async_copy(table.at[sd1.at[0]], rows, sem).wait()
            pltpu.async_copy(rows, acc.at[sd1.at[1]], sem2,
                             add=True).wait()
            p0.wait()
            return carry

        lax.fori_loop(0, K // 2, pipe, 0)
        plsc.subcore_barrier()

        for j in range(WB):
            off = base + j * 128
            pltpu.sync_copy(acc.at[pl.ds(off, 128)], rows)
            pltpu.sync_copy(rows, acc_out.at[c, pl.ds(off, 128)])

    return pl.kernel(
        body,
        out_type=[jax.ShapeDtypeStruct((NC, N_PAD, D), F32)],
        mesh=mesh,
        scratch_types=[
            pltpu.VMEM_SHARED((N_PAD, D), F32),
            pltpu.VMEM((2, LANE), jnp.int32),
            pltpu.VMEM((2, LANE), jnp.int32),
            pltpu.VMEM((LANE, D), F32),
            pltpu.SemaphoreType.DMA,
            pltpu.SemaphoreType.DMA,
            pltpu.SemaphoreType.DMA,
            pltpu.SemaphoreType.DMA,
        ])


def _sc_count():
    mesh = plsc.VectorSubcoreMesh(core_axis_name="c", subcore_axis_name="s")

    def body(dst3, z128, o128, cnt_out, acc, dst_v, rows, sem2):
        c = lax.axis_index("c")
        s = lax.axis_index("s")
        wid = c * NS + s
        base = s * ROWS_PER_TILE

        pltpu.sync_copy(z128, rows)
        for j in range(WB):
            pltpu.sync_copy(rows, acc.at[pl.ds(base + j * 128, 128)])
        plsc.subcore_barrier()

        pltpu.sync_copy(o128, rows)

        def chunk(cc, carry):
            pltpu.sync_copy(dst3.at[wid, cc], dst_v)
            pltpu.async_copy(rows, acc.at[dst_v], sem2, add=True).wait()
            return carry

        lax.fori_loop(0, K, chunk, 0)
        plsc.subcore_barrier()

        for j in range(WB):
            off = base + j * 128
            pltpu.sync_copy(acc.at[pl.ds(off, 128)], rows)
            pltpu.sync_copy(rows, cnt_out.at[c, pl.ds(off, 128)])

    return pl.kernel(
        body,
        out_type=[jax.ShapeDtypeStruct((NC, N_PAD, D), F32)],
        mesh=mesh,
        scratch_types=[
            pltpu.VMEM_SHARED((N_PAD, D), F32),
            pltpu.VMEM((LANE,), jnp.int32),
            pltpu.VMEM((LANE, D), F32),
            pltpu.SemaphoreType.DMA,
        ])


def _proj2(x, wa, wb, bias):
    blk = 1000

    def body(x_ref, wa_ref, wb_ref, b_ref, y_ref, z_ref):
        xb = x_ref[...]
        y_ref[...] = jnp.dot(xb, wa_ref[...], preferred_element_type=F32)
        z_ref[...] = jnp.dot(xb, wb_ref[...],
                             preferred_element_type=F32) + b_ref[...]

    return pl.pallas_call(
        body,
        grid=(N // blk,),
        in_specs=[
            pl.BlockSpec((blk, D), lambda i: (i, 0)),
            pl.BlockSpec((D, D), lambda i: (0, 0)),
            pl.BlockSpec((D, D), lambda i: (0, 0)),
            pl.BlockSpec((1, D), lambda i: (0, 0)),
        ],
        out_specs=[
            pl.BlockSpec((blk, D), lambda i: (i, 0)),
            pl.BlockSpec((blk, D), lambda i: (i, 0)),
        ],
        out_shape=[
            jax.ShapeDtypeStruct((N, D), F32),
            jax.ShapeDtypeStruct((N, D), F32),
        ],
    )(x, wa, wb, bias)


def _combine_project(z, accp, cntp, wa, wb, bias):
    blk = 1000

    def body(z_ref, a_ref, c_ref, wa_ref, wb_ref, b_ref, y_ref, zo_ref):
        agg = a_ref[0] + a_ref[1]
        cnt = (c_ref[0] + c_ref[1])[:, 0:1]
        o = agg / jnp.maximum(cnt, 1.0) + z_ref[...]
        nrm = jnp.sqrt(jnp.sum(o * o, axis=-1, keepdims=True))
        h = jnp.maximum(o / jnp.maximum(nrm, 1e-12), 0.0)
        y_ref[...] = jnp.dot(h, wa_ref[...], preferred_element_type=F32)
        zo_ref[...] = jnp.dot(h, wb_ref[...],
                              preferred_element_type=F32) + b_ref[...]

    return pl.pallas_call(
        body,
        grid=(N // blk,),
        in_specs=[
            pl.BlockSpec((blk, D), lambda i: (i, 0)),
            pl.BlockSpec((2, blk, D), lambda i: (0, i, 0)),
            pl.BlockSpec((2, blk, D), lambda i: (0, i, 0)),
            pl.BlockSpec((D, D), lambda i: (0, 0)),
            pl.BlockSpec((D, D), lambda i: (0, 0)),
            pl.BlockSpec((1, D), lambda i: (0, 0)),
        ],
        out_specs=[
            pl.BlockSpec((blk, D), lambda i: (i, 0)),
            pl.BlockSpec((blk, D), lambda i: (i, 0)),
        ],
        out_shape=[
            jax.ShapeDtypeStruct((N, D), F32),
            jax.ShapeDtypeStruct((N, D), F32),
        ],
    )(z, accp, cntp, wa, wb, bias)


def _combine_final(z, accp, cntp):
    blk = 1000

    def body(z_ref, a_ref, c_ref, o_ref):
        agg = a_ref[0] + a_ref[1]
        cnt = (c_ref[0] + c_ref[1])[:, 0:1]
        o = agg / jnp.maximum(cnt, 1.0) + z_ref[...]
        nrm = jnp.sqrt(jnp.sum(o * o, axis=-1, keepdims=True))
        o_ref[...] = o / jnp.maximum(nrm, 1e-12)

    return pl.pallas_call(
        body,
        grid=(N // blk,),
        in_specs=[
            pl.BlockSpec((blk, D), lambda i: (i, 0)),
            pl.BlockSpec((2, blk, D), lambda i: (0, i, 0)),
            pl.BlockSpec((2, blk, D), lambda i: (0, i, 0)),
        ],
        out_specs=pl.BlockSpec((blk, D), lambda i: (i, 0)),
        out_shape=jax.ShapeDtypeStruct((N, D), F32),
    )(z, accp, cntp)


def kernel(x, edge_index, W1l, b1l, W1r, W2l, b2l, W2r):
    src = edge_index[0]
    dst = edge_index[1]
    pad = E_PAD - E
    src3 = jnp.concatenate(
        [src, jnp.zeros((pad,), jnp.int32)]).reshape(NC * NS, K, 1, LANE)
    dst3 = jnp.concatenate(
        [dst, jnp.full((pad,), N_PAD - 1, jnp.int32)]).reshape(NC * NS, K, 1,
                                                               LANE)
    sd3 = jnp.concatenate([src3, dst3], axis=2)
    dummy = jnp.concatenate(
        [jnp.zeros((NC * NS, 1, 1, LANE), jnp.int32),
         jnp.full((NC * NS, 1, 1, LANE), N_PAD - 1, jnp.int32)], axis=2)
    sd3 = jnp.concatenate([sd3, dummy], axis=1)
    dst3 = dst3.reshape(NC * NS, K, LANE)
    z128 = jnp.zeros((128, D), F32)

    sc = _sc_scatter()
    scc = _sc_count()
    o128 = jnp.ones((128, D), F32)

    y1, z1 = _proj2(x, W1l.T, W1r.T, b1l.reshape(1, D))
    (acc1,) = sc(y1, sd3, z128)
    (cntp,) = scc(dst3, z128, o128)
    y2, z2b = _combine_project(z1, acc1, cntp, W2l.T, W2r.T, b2l.reshape(1, D))
    (acc2,) = sc(y2, sd3, z128)
    return _combine_final(z2b, acc2, cntp)


# speedup vs baseline: 1.4888x; 1.4888x over previous
import jax
import jax.numpy as jnp
from jax import lax
from jax.experimental import pallas as pl
from jax.experimental.pallas import tpu as pltpu
from jax.experimental.pallas import tpu_sc as plsc

N = 10000
D = 128
E = 320000

NC = 2
NS = 16
LANE = 128
N_PAD = 10240
ROWS_PER_TILE = N_PAD // NS
WB = ROWS_PER_TILE // 128
K = -(-E // (NC * NS * LANE))
E_PAD = NC * NS * K * LANE
F32 = jnp.float32


def _sc_scatter():
    mesh = plsc.VectorSubcoreMesh(core_axis_name="c", subcore_axis_name="s")

    def body(table, src3, z128, acc_out, acc, sdall, rows, sem, sem2):
        c = lax.axis_index("c")
        s = lax.axis_index("s")
        wid = c * NS + s
        base = s * ROWS_PER_TILE

        pltpu.sync_copy(src3.at[wid], sdall)
        pltpu.sync_copy(z128, rows)
        for j in range(WB):
            pltpu.sync_copy(rows, acc.at[pl.ds(base + j * 128, 128)])
        plsc.subcore_barrier()

        def chunk(cc, carry):
            pltpu.async_copy(table.at[sdall.at[cc, 0]], rows, sem).wait()
            pltpu.async_copy(rows, acc.at[sdall.at[cc, 1]], sem2,
                             add=True).wait()
            return carry

        lax.fori_loop(0, K, chunk, 0)
        plsc.subcore_barrier()

        for j in range(WB):
            off = base + j * 128
            pltpu.sync_copy(acc.at[pl.ds(off, 128)], rows)
            pltpu.sync_copy(rows, acc_out.at[c, pl.ds(off, 128)])

    return pl.kernel(
        body,
        out_type=[jax.ShapeDtypeStruct((NC, N_PAD, D), F32)],
        mesh=mesh,
        scratch_types=[
            pltpu.VMEM_SHARED((N_PAD, D), F32),
            pltpu.VMEM((K, 2, LANE), jnp.int32),
            pltpu.VMEM((LANE, D), F32),
            pltpu.SemaphoreType.DMA,
            pltpu.SemaphoreType.DMA,
        ])


def _sc_count():
    mesh = plsc.VectorSubcoreMesh(core_axis_name="c", subcore_axis_name="s")

    def body(dst3, z128, o128, cnt_out, acc, dall, rows, sem2):
        c = lax.axis_index("c")
        s = lax.axis_index("s")
        wid = c * NS + s
        base = s * ROWS_PER_TILE

        pltpu.sync_copy(z128, rows)
        for j in range(WB):
            pltpu.sync_copy(rows, acc.at[pl.ds(base + j * 128, 128)])
        plsc.subcore_barrier()

        pltpu.sync_copy(o128, rows)
        pltpu.sync_copy(dst3.at[wid], dall)

        def chunk(cc, carry):
            pltpu.async_copy(rows, acc.at[dall.at[cc]], sem2,
                             add=True).wait()
            return carry

        lax.fori_loop(0, K, chunk, 0)
        plsc.subcore_barrier()

        for j in range(WB):
            off = base + j * 128
            pltpu.sync_copy(acc.at[pl.ds(off, 128)], rows)
            pltpu.sync_copy(rows, cnt_out.at[c, pl.ds(off, 128)])

    return pl.kernel(
        body,
        out_type=[jax.ShapeDtypeStruct((NC, N_PAD, D), F32)],
        mesh=mesh,
        scratch_types=[
            pltpu.VMEM_SHARED((N_PAD, D), F32),
            pltpu.VMEM((K, LANE), jnp.int32),
            pltpu.VMEM((LANE, D), F32),
            pltpu.SemaphoreType.DMA,
        ])


def _proj2(x, wa, wb, bias):
    blk = 1000

    def body(x_ref, wa_ref, wb_ref, b_ref, y_ref, z_ref):
        xb = x_ref[...]
        y_ref[...] = jnp.dot(xb, wa_ref[...], preferred_element_type=F32)
        z_ref[...] = jnp.dot(xb, wb_ref[...],
                             preferred_element_type=F32) + b_ref[...]

    return pl.pallas_call(
        body,
        grid=(N // blk,),
        in_specs=[
            pl.BlockSpec((blk, D), lambda i: (i, 0)),
            pl.BlockSpec((D, D), lambda i: (0, 0)),
            pl.BlockSpec((D, D), lambda i: (0, 0)),
            pl.BlockSpec((1, D), lambda i: (0, 0)),
        ],
        out_specs=[
            pl.BlockSpec((blk, D), lambda i: (i, 0)),
            pl.BlockSpec((blk, D), lambda i: (i, 0)),
        ],
        out_shape=[
            jax.ShapeDtypeStruct((N, D), F32),
            jax.ShapeDtypeStruct((N, D), F32),
        ],
    )(x, wa, wb, bias)


def _combine_project(z, accp, cntp, wa, wb, bias):
    blk = 1000

    def body(z_ref, a_ref, c_ref, wa_ref, wb_ref, b_ref, y_ref, zo_ref):
        agg = a_ref[0] + a_ref[1]
        cnt = (c_ref[0] + c_ref[1])[:, 0:1]
        o = agg / jnp.maximum(cnt, 1.0) + z_ref[...]
        nrm = jnp.sqrt(jnp.sum(o * o, axis=-1, keepdims=True))
        h = jnp.maximum(o / jnp.maximum(nrm, 1e-12), 0.0)
        y_ref[...] = jnp.dot(h, wa_ref[...], preferred_element_type=F32)
        zo_ref[...] = jnp.dot(h, wb_ref[...],
                              preferred_element_type=F32) + b_ref[...]

    return pl.pallas_call(
        body,
        grid=(N // blk,),
        in_specs=[
            pl.BlockSpec((blk, D), lambda i: (i, 0)),
            pl.BlockSpec((2, blk, D), lambda i: (0, i, 0)),
            pl.BlockSpec((2, blk, D), lambda i: (0, i, 0)),
            pl.BlockSpec((D, D), lambda i: (0, 0)),
            pl.BlockSpec((D, D), lambda i: (0, 0)),
            pl.BlockSpec((1, D), lambda i: (0, 0)),
        ],
        out_specs=[
            pl.BlockSpec((blk, D), lambda i: (i, 0)),
            pl.BlockSpec((blk, D), lambda i: (i, 0)),
        ],
        out_shape=[
            jax.ShapeDtypeStruct((N, D), F32),
            jax.ShapeDtypeStruct((N, D), F32),
        ],
    )(z, accp, cntp, wa, wb, bias)


def _combine_final(z, accp, cntp):
    blk = 1000

    def body(z_ref, a_ref, c_ref, o_ref):
        agg = a_ref[0] + a_ref[1]
        cnt = (c_ref[0] + c_ref[1])[:, 0:1]
        o = agg / jnp.maximum(cnt, 1.0) + z_ref[...]
        nrm = jnp.sqrt(jnp.sum(o * o, axis=-1, keepdims=True))
        o_ref[...] = o / jnp.maximum(nrm, 1e-12)

    return pl.pallas_call(
        body,
        grid=(N // blk,),
        in_specs=[
            pl.BlockSpec((blk, D), lambda i: (i, 0)),
            pl.BlockSpec((2, blk, D), lambda i: (0, i, 0)),
            pl.BlockSpec((2, blk, D), lambda i: (0, i, 0)),
        ],
        out_specs=pl.BlockSpec((blk, D), lambda i: (i, 0)),
        out_shape=jax.ShapeDtypeStruct((N, D), F32),
    )(z, accp, cntp)


def kernel(x, edge_index, W1l, b1l, W1r, W2l, b2l, W2r):
    src = edge_index[0]
    dst = edge_index[1]
    pad = E_PAD - E
    src3 = jnp.concatenate(
        [src, jnp.zeros((pad,), jnp.int32)]).reshape(NC * NS, K, 1, LANE)
    dst3 = jnp.concatenate(
        [dst, jnp.full((pad,), N_PAD - 1, jnp.int32)]).reshape(NC * NS, K, 1,
                                                               LANE)
    sd3 = jnp.concatenate([src3, dst3], axis=2)
    dst3 = dst3.reshape(NC * NS, K, LANE)
    z128 = jnp.zeros((128, D), F32)

    sc = _sc_scatter()
    scc = _sc_count()
    o128 = jnp.ones((128, D), F32)

    y1, z1 = _proj2(x, W1l.T, W1r.T, b1l.reshape(1, D))
    (acc1,) = sc(y1, sd3, z128)
    (cntp,) = scc(dst3, z128, o128)
    y2, z2b = _combine_project(z1, acc1, cntp, W2l.T, W2r.T, b2l.reshape(1, D))
    (acc2,) = sc(y2, sd3, z128)
    return _combine_final(z2b, acc2, cntp)
